# baseline (speedup 1.0000x reference)
import jax
import jax.numpy as jnp
from jax.experimental import pallas as pl
from jax.experimental.pallas import tpu as pltpu

BM = 256
N = 4096
NB = N // BM


def _dot(a, b):
    return jnp.dot(a, b, preferred_element_type=jnp.float32)


def _b16(x):
    return x.astype(jnp.bfloat16)


def _body(u_adj, inner, v_adj, u_attr, v_attr,
          Wn_v, Ws_v, b_v, Wn_u, Ws_u, b_u, W_in1, b_in1,
          Wn_v2, Ws_v2, b_v2, Wn_u2, Ws_u2, b_u2, W_in2, b_in2,
          u_emb2, v_emb2, rating,
          PSu, PSv, MID, VT, IC, DB, sem):
    i = pl.program_id(0)
    p = i // NB
    r = i % NB
    row = pl.ds(r * BM, BM)
    slot = jax.lax.rem(i, 4)

    def issue(j):
        q = j // NB
        rows = pl.ds((j % NB) * BM, BM)
        dst = DB.at[jax.lax.rem(j, 4)]
        s = sem.at[jax.lax.rem(j, 4)]

        @pl.when(jnp.logical_or(q == 0, q == 3))
        def _():
            pltpu.make_async_copy(u_adj.at[rows, :], dst, s).start()

        @pl.when(q == 1)
        def _():
            pltpu.make_async_copy(inner.at[rows, :], dst, s).start()

        @pl.when(q == 2)
        def _():
            pltpu.make_async_copy(v_adj.at[rows, :], dst, s).start()

    @pl.when(i == 0)
    def _():
        issue(0)
        issue(1)
        issue(2)
        issue(3)
        PSu[:, 0:64] = _b16(_dot(v_attr[...], _b16(Wn_u[...])))
        PSv[:, 64:128] = _b16(_dot(v_attr[...], _b16(Ws_v[...])) + b_v[...])

    @pl.when(jnp.logical_and(i + 3 < 4 * NB, i > 0))
    def _():
        issue(i + 3)

    @pl.when(i < 4 * NB)
    def _():
        pltpu.make_async_copy(DB.at[slot], DB.at[slot], sem.at[slot]).wait()

    @pl.when(p == 0)
    def _():
        su = _dot(u_attr[...], _b16(Ws_u[...])) + b_u[...]
        PSv[row, 0:64] = _b16(_dot(u_attr[...], _b16(Wn_v[...])))
        a = _b16(DB[slot])
        u1 = jnp.maximum(_dot(a, PSu[:, 0:64]) + su, 0.0)
        MID[row, 0:64] = _b16(_dot(u1, W_in1[...]))

    @pl.when(p == 1)
    def _():
        a = _b16(DB[slot])
        IC[row, :] = a
        t = jnp.maximum(_dot(a, MID[:, 0:64]) + b_in1[...], 0.0)
        MID[row, 64:72] = _b16(_dot(t, Wn_v2[...]))
        MID[row, 72:80] = _b16(_dot(t, Ws_u2[...]) + b_u2[...])

    @pl.when(p == 2)
    def _():
        a = _b16(DB[slot])
        ve = jnp.maximum(
            _dot(a, PSv[:, 0:64]) + PSv[row, 64:128].astype(jnp.float32),
            0.0)
        ve2 = jnp.maximum(
            _dot(a, MID[:, 64:72]) + _dot(ve, Ws_v2[...])
            + b_v2[...], 0.0)
        v_emb2[...] = ve2
        VT[:, row] = _b16(ve2).T
        MID[row, 80:88] = _b16(_dot(ve, Wn_u2[...]))

    @pl.when(p == 3)
    def _():
        a = _b16(DB[slot])
        u2a = jnp.maximum(
            _dot(a, MID[:, 80:88]) + MID[row, 72:80].astype(jnp.float32),
            0.0)
        MID[row, 88:96] = _b16(_dot(u2a, W_in2[...]))

    @pl.when(p == 4)
    def _():
        t = jnp.maximum(
            _dot(IC[row, :], MID[:, 88:96]) + b_in2[...], 0.0)
        u_emb2[...] = t
        rating[...] = jnp.tanh(_dot(_b16(t), VT[...]))


def _im_p2out(i):
    r = i % NB
    return (jnp.where(i < 2 * NB, 0, jnp.where(i < 3 * NB, r, NB - 1)), 0)


def _im_p4out(i):
    r = i % NB
    return (jnp.where(i < 4 * NB, 0, r), 0)


def _full(shape):
    return pl.BlockSpec(shape, lambda i: (0,) * len(shape))


_ANY = pl.BlockSpec(memory_space=pl.ANY)


@jax.jit
def kernel(u_attr, v_attr, u_adj, v_adj, u_adj_inner,
           Wn_v, Ws_v, b_v, Wn_u, Ws_u, b_u, W_in1, b_in1,
           Wn_v2, Ws_v2, b_v2, Wn_u2, Ws_u2, b_u2, W_in2, b_in2):
    f32 = jnp.float32
    bf16 = jnp.bfloat16
    H = Wn_v.shape[1]
    O = Wn_v2.shape[1]
    DU = u_attr.shape[1]
    DV = v_attr.shape[1]
    b_v = b_v.reshape(1, H)
    b_u = b_u.reshape(1, H)
    b_in1 = b_in1.reshape(1, H)
    b_v2 = b_v2.reshape(1, O)
    b_u2 = b_u2.reshape(1, O)
    b_in2 = b_in2.reshape(1, O)
    u_attr = u_attr.astype(bf16)
    v_attr = v_attr.astype(bf16)

    u_emb2, v_emb2, rating = pl.pallas_call(
        _body,
        grid=(5 * NB,),
        in_specs=[
            _ANY, _ANY, _ANY,
            pl.BlockSpec((BM, DU),
                         lambda i: (jnp.where(i < NB, i % NB, NB - 1), 0)),
            _full((N, DV)),
            _full((DU, H)), _full((DV, H)), _full((1, H)),
            _full((DV, H)), _full((DU, H)), _full((1, H)),
            _full((H, H)), _full((1, H)),
            _full((H, O)), _full((H, O)), _full((1, O)),
            _full((H, O)), _full((H, O)), _full((1, O)),
            _full((O, O)), _full((1, O)),
        ],
        out_specs=[
            pl.BlockSpec((BM, O), _im_p4out),
            pl.BlockSpec((BM, O), _im_p2out),
            pl.BlockSpec((BM, N), _im_p4out),
        ],
        out_shape=[
            jax.ShapeDtypeStruct((N, O), f32),
            jax.ShapeDtypeStruct((N, O), f32),
            jax.ShapeDtypeStruct((N, N), f32),
        ],
        scratch_shapes=[
            pltpu.VMEM((N, 128), bf16),
            pltpu.VMEM((N, 128), bf16),
            pltpu.VMEM((N, 128), bf16),
            pltpu.VMEM((O, N), bf16),
            pltpu.VMEM((N, N), bf16),
            pltpu.VMEM((4, BM, N), f32),
            pltpu.SemaphoreType.DMA((4,)),
        ],
        compiler_params=pltpu.CompilerParams(
            vmem_limit_bytes=64 * 1024 * 1024),
    )(u_adj, u_adj_inner, v_adj, u_attr, v_attr,
      Wn_v, Ws_v, b_v, Wn_u, Ws_u, b_u, W_in1, b_in1,
      Wn_v2, Ws_v2, b_v2, Wn_u2, Ws_u2, b_u2, W_in2, b_in2)

    return (u_emb2, v_emb2, rating)


# speedup vs baseline: 1.0470x; 1.0470x over previous
import jax
import jax.numpy as jnp
from jax.experimental import pallas as pl
from jax.experimental.pallas import tpu as pltpu

BM = 256
N = 4096
NB = N // BM


def _dot(a, b):
    return jnp.dot(a, b, preferred_element_type=jnp.float32)


def _b16(x):
    return x.astype(jnp.bfloat16)


def _body(u_adj, inner, v_adj, u_attr, v_attr, W1, W2, W3, W4, B1, B2,
          u_emb2, v_emb2, rating,
          PSu, PSv, MID, VT, IC, DB, sem):
    i = pl.program_id(0)
    p = i // NB
    r = i % NB
    row = pl.ds(r * BM, BM)
    slot = jax.lax.rem(i, 3)

    def issue(j):
        q = j // NB
        rows = pl.ds((j % NB) * BM, BM)
        dst = DB.at[jax.lax.rem(j, 3)]
        s = sem.at[jax.lax.rem(j, 3)]

        @pl.when(jnp.logical_or(q == 0, q == 3))
        def _():
            pltpu.make_async_copy(u_adj.at[rows, :], dst, s).start()

        @pl.when(q == 1)
        def _():
            pltpu.make_async_copy(inner.at[rows, :], dst, s).start()

        @pl.when(q == 2)
        def _():
            pltpu.make_async_copy(v_adj.at[rows, :], dst, s).start()

    @pl.when(i == 0)
    def _():
        issue(0)
        issue(1)
        issue(2)
        PSu[:, 0:64] = _dot(v_attr[...], W2[:, 0:64])
        PSv[:, 64:128] = _dot(v_attr[...], W1[:, 64:128]) + B1[0:1, :]

    @pl.when(jnp.logical_and(i + 2 < 4 * NB, i > 0))
    def _():
        issue(i + 2)

    @pl.when(i < 4 * NB)
    def _():
        pltpu.make_async_copy(DB.at[slot], DB.at[slot], sem.at[slot]).wait()

    @pl.when(p == 0)
    def _():
        su = _dot(u_attr[...], W2[:, 64:128]) + B1[1:2, :]
        PSv[row, 0:64] = _dot(u_attr[...], W1[:, 0:64])
        a = _b16(DB[slot])
        u1 = jnp.maximum(_dot(a, _b16(PSu[:, 0:64])) + su, 0.0)
        MID[row, 0:64] = _dot(u1, W3[:, 0:64])

    @pl.when(p == 1)
    def _():
        a = _b16(DB[slot])
        IC[row, :] = a
        t = jnp.maximum(_dot(a, _b16(MID[:, 0:64])) + B1[2:3, :], 0.0)
        MID[row, 64:72] = _dot(t, W3[:, 64:72])
        MID[row, 72:80] = _dot(t, W3[:, 88:96]) + B2[1:2, :]

    @pl.when(p == 2)
    def _():
        a = _b16(DB[slot])
        ve = jnp.maximum(
            _dot(a, _b16(PSv[:, 0:64])) + PSv[row, 64:128], 0.0)
        ve2 = jnp.maximum(
            _dot(a, _b16(MID[:, 64:72])) + _dot(ve, W3[:, 72:80])
            + B2[0:1, :], 0.0)
        v_emb2[...] = ve2
        VT[:, row] = ve2.T
        MID[row, 80:88] = _dot(ve, W3[:, 80:88])

    @pl.when(p == 3)
    def _():
        a = _b16(DB[slot])
        u2a = jnp.maximum(
            _dot(a, _b16(MID[:, 80:88])) + MID[row, 72:80], 0.0)
        MID[row, 88:96] = _dot(u2a, W4[...])

    @pl.when(p == 4)
    def _():
        t = jnp.maximum(
            _dot(IC[row, :], _b16(MID[:, 88:96])) + B2[2:3, :], 0.0)
        u_emb2[...] = t
        rating[...] = jnp.tanh(_dot(_b16(t), _b16(VT[...])))


def _im_p2out(i):
    r = i % NB
    return (jnp.where(i < 2 * NB, 0, jnp.where(i < 3 * NB, r, NB - 1)), 0)


def _im_p4out(i):
    r = i % NB
    return (jnp.where(i < 4 * NB, 0, r), 0)


def _full(shape):
    return pl.BlockSpec(shape, lambda i: (0,) * len(shape))


_ANY = pl.BlockSpec(memory_space=pl.ANY)


@jax.jit
def kernel(u_attr, v_attr, u_adj, v_adj, u_adj_inner,
           Wn_v, Ws_v, b_v, Wn_u, Ws_u, b_u, W_in1, b_in1,
           Wn_v2, Ws_v2, b_v2, Wn_u2, Ws_u2, b_u2, W_in2, b_in2):
    f32 = jnp.float32
    bf16 = jnp.bfloat16
    H = Wn_v.shape[1]
    O = Wn_v2.shape[1]
    DU = u_attr.shape[1]
    DV = v_attr.shape[1]
    W1 = jnp.concatenate([Wn_v, Ws_v], axis=1)
    W2 = jnp.concatenate([Wn_u, Ws_u], axis=1)
    W3 = jnp.concatenate([W_in1, Wn_v2, Ws_v2, Wn_u2, Ws_u2],
                         axis=1)
    W4 = W_in2
    B1 = jnp.stack([b_v, b_u, b_in1])
    B2 = jnp.stack([b_v2, b_u2, b_in2])

    u_emb2, v_emb2, rating = pl.pallas_call(
        _body,
        grid=(5 * NB,),
        in_specs=[
            _ANY, _ANY, _ANY,
            pl.BlockSpec((BM, DU),
                         lambda i: (jnp.where(i < NB, i % NB, NB - 1), 0)),
            _full((N, DV)),
            _full((DU, 2 * H)), _full((DV, 2 * H)),
            _full((H, H + 4 * O)), _full((O, O)),
            _full((3, H)), _full((3, O)),
        ],
        out_specs=[
            pl.BlockSpec((BM, O), _im_p4out),
            pl.BlockSpec((BM, O), _im_p2out),
            pl.BlockSpec((BM, N), _im_p4out),
        ],
        out_shape=[
            jax.ShapeDtypeStruct((N, O), f32),
            jax.ShapeDtypeStruct((N, O), f32),
            jax.ShapeDtypeStruct((N, N), f32),
        ],
        scratch_shapes=[
            pltpu.VMEM((N, 128), f32),
            pltpu.VMEM((N, 128), f32),
            pltpu.VMEM((N, 128), f32),
            pltpu.VMEM((O, N), f32),
            pltpu.VMEM((N, N), bf16),
            pltpu.VMEM((3, BM, N), f32),
            pltpu.SemaphoreType.DMA((3,)),
        ],
        compiler_params=pltpu.CompilerParams(
            vmem_limit_bytes=64 * 1024 * 1024),
    )(u_adj, u_adj_inner, v_adj, u_attr, v_attr,
      W1, W2, W3, W4, B1, B2)

    return (u_emb2, v_emb2, rating)
